# Initial kernel scaffold; baseline (speedup 1.0000x reference)
import jax
import jax.numpy as jnp
from jax.experimental import pallas as pl


def kernel(distances):
    raise NotImplementedError("write your pallas kernel here")


# speedup vs baseline: 6.0046x; 6.0046x over previous
import functools

import numpy as np
import jax
import jax.numpy as jnp
from jax import lax
from jax.experimental import pallas as pl
from jax.experimental.pallas import tpu as pltpu
from jax.experimental.pallas import tpu_sc as plsc

_ROWS = 64
_N = 65536
_B = 2048
_XMIN = -6.9078
_XMAX = 6.9078
_INVW = _B / (_XMAX - _XMIN)
_NC = 2
_NS = 16
_CH = 8192

_yi = -np.log(1.0 - np.arange(_N, dtype=np.float64) / _N)
_Y_TABLE = np.concatenate([[0.0], np.cumsum(_yi)]).astype(np.float32)


def _log_body(d_ref, o_ref):
    o_ref[...] = jnp.log(d_ref[...])


def _log_all(dflat):
    return pl.pallas_call(
        _log_body,
        out_shape=jax.ShapeDtypeStruct(dflat.shape, jnp.float32),
        grid=(8, 4),
        in_specs=[pl.BlockSpec((8, 2 * _N // 4), lambda i, j: (i, j))],
        out_specs=pl.BlockSpec((8, 2 * _N // 4), lambda i, j: (i, j)),
    )(dflat)


def _sc_body(L_hbm, Y_hbm, num_hbm, den_hbm,
             chunk, s_cnt, s_sum, cnts, sums, cexc, cinc, yc, yt, outv, sem):
    wid = lax.axis_index("s") * _NC + lax.axis_index("c")
    lane = lax.iota(jnp.int32, 16)
    lane_base = lane * _B
    ones = jnp.ones((16,), jnp.int32)

    for rr in range(2):
        r = wid * 2 + rr

        def zbody(i, _):
            s_cnt[pl.ds(i * 16, 16)] = jnp.zeros((16,), jnp.int32)
            s_sum[pl.ds(i * 16, 16)] = jnp.zeros((16,), jnp.float32)
            return 0
        lax.fori_loop(0, _B, zbody, 0)

        def chunk_loop(c, den_acc):
            pltpu.sync_copy(L_hbm.at[r, pl.ds(c * _CH, _CH)], chunk)

            def vloop(v, dacc):
                idx_e = v * 32 + 2 * lane
                le = plsc.load_gather(chunk, [idx_e])
                lo = plsc.load_gather(chunk, [idx_e + 1])
                x = lo - le
                dacc = dacc + x * x
                binf = (x - _XMIN) * _INVW
                b = jnp.clip(binf.astype(jnp.int32), 0, _B - 1)
                addr = lane_base + b
                plsc.addupdate_scatter(s_cnt, [addr], ones)
                plsc.addupdate_scatter(s_sum, [addr], x)
                return dacc

            return lax.fori_loop(0, _CH // 32, vloop, den_acc)

        den_acc = lax.fori_loop(0, 2 * _N // _CH, chunk_loop,
                                jnp.zeros((16,), jnp.float32))

        def red(j, _):
            off = j * 16
            ci = s_cnt[pl.ds(off, 16)]
            si = s_sum[pl.ds(off, 16)]
            for l in range(1, 16):
                ci = ci + s_cnt[pl.ds(l * _B + off, 16)]
                si = si + s_sum[pl.ds(l * _B + off, 16)]
            cnts[pl.ds(off, 16)] = ci
            sums[pl.ds(off, 16)] = si
            return 0
        lax.fori_loop(0, _B // 16, red, 0)

        def scan(j, carry):
            v = cnts[pl.ds(j * 16, 16)]
            inc = plsc.cumsum(v) + carry
            row_i = j // 8
            col = (j % 8) * 16
            cexc[row_i, pl.ds(col, 16)] = inc - v
            cinc[row_i, pl.ds(col, 16)] = inc
            return carry + jnp.sum(v)
        lax.fori_loop(0, _B // 16, scan, jnp.int32(0))

        handles = []
        for i in range(16):
            handles.append(pltpu.async_copy(Y_hbm.at[cexc.at[i]], yc.at[i], sem))
            handles.append(pltpu.async_copy(Y_hbm.at[cinc.at[i]], yt.at[i], sem))
        for h in handles:
            h.wait()

        def comb(j, acc):
            row_i = j // 8
            col = (j % 8) * 16
            t = cnts[pl.ds(j * 16, 16)]
            s = sums[pl.ds(j * 16, 16)]
            a = yc[row_i, pl.ds(col, 16)]
            bb = yt[row_i, pl.ds(col, 16)]
            tf = t.astype(jnp.float32)
            avg = s / jnp.maximum(tf, 1.0)
            return acc + jnp.where(t > 0, avg * (bb - a), 0.0)
        acc = lax.fori_loop(0, _B // 16, comb, jnp.zeros((16,), jnp.float32))

        num_s = jnp.sum(acc)
        den_s = jnp.sum(den_acc)
        outv[...] = jnp.broadcast_to(num_s, (16,))
        pltpu.sync_copy(outv, num_hbm.at[r])
        outv[...] = jnp.broadcast_to(den_s, (16,))
        pltpu.sync_copy(outv, den_hbm.at[r])


def _sc_estimate(L, Y):
    mesh = plsc.VectorSubcoreMesh(core_axis_name="c", subcore_axis_name="s")
    f = pl.kernel(
        _sc_body,
        out_type=[
            jax.ShapeDtypeStruct((_ROWS, 16), jnp.float32),
            jax.ShapeDtypeStruct((_ROWS, 16), jnp.float32),
        ],
        mesh=mesh,
        compiler_params=pltpu.CompilerParams(needs_layout_passes=False),
        scratch_types=[
            pltpu.VMEM((_CH,), jnp.float32),
            pltpu.VMEM((_B * 16,), jnp.int32),
            pltpu.VMEM((_B * 16,), jnp.float32),
            pltpu.VMEM((_B,), jnp.int32),
            pltpu.VMEM((_B,), jnp.float32),
            pltpu.VMEM((16, _B // 16), jnp.int32),
            pltpu.VMEM((16, _B // 16), jnp.int32),
            pltpu.VMEM((16, _B // 16), jnp.float32),
            pltpu.VMEM((16, _B // 16), jnp.float32),
            pltpu.VMEM((16,), jnp.float32),
            pltpu.SemaphoreType.DMA,
        ],
    )
    return f(L, Y)


def kernel(distances):
    dflat = distances.reshape(_ROWS, 2 * _N)
    L = _log_all(dflat)
    Y = jnp.asarray(_Y_TABLE)
    num, den = _sc_estimate(L, Y)
    return num[:, 0] / den[:, 0]


# speedup vs baseline: 6.3121x; 1.0512x over previous
import functools

import numpy as np
import jax
import jax.numpy as jnp
from jax import lax
from jax.experimental import pallas as pl
from jax.experimental.pallas import tpu as pltpu
from jax.experimental.pallas import tpu_sc as plsc

_ROWS = 64
_N = 65536
_B = 2048
_XMIN = -6.9078
_XMAX = 6.9078
_INVW = _B / (_XMAX - _XMIN)
_NC = 2
_NS = 16
_CH = 8192

_yi = -np.log(1.0 - np.arange(_N, dtype=np.float64) / _N)
_Y_TABLE = np.concatenate([[0.0], np.cumsum(_yi)]).astype(np.float32)


def _log_body(d_ref, o_ref):
    o_ref[...] = jnp.log(d_ref[...])


def _log_all(dflat):
    return pl.pallas_call(
        _log_body,
        out_shape=jax.ShapeDtypeStruct(dflat.shape, jnp.float32),
        grid=(8, 4),
        in_specs=[pl.BlockSpec((8, 2 * _N // 4), lambda i, j: (i, j))],
        out_specs=pl.BlockSpec((8, 2 * _N // 4), lambda i, j: (i, j)),
    )(dflat)


def _sc_body(L_hbm, Y_hbm, num_hbm, den_hbm,
             chunk0, chunk1, s_cnt, s_sum, cnts, sums, cexc, cinc, yc, yt,
             outv, sem0, sem1, sem):
    wid = lax.axis_index("s") * _NC + lax.axis_index("c")
    lane = lax.iota(jnp.int32, 16)
    lane_base = lane * _B
    ones = jnp.ones((16,), jnp.int32)

    for rr in range(2):
        r = wid * 2 + rr

        def zbody(i, _):
            s_cnt[pl.ds(i * 16, 16)] = jnp.zeros((16,), jnp.int32)
            s_sum[pl.ds(i * 16, 16)] = jnp.zeros((16,), jnp.float32)
            return 0
        lax.fori_loop(0, _B, zbody, 0, unroll=8)

        n_chunks = 2 * _N // _CH
        bufs = (chunk0, chunk1)
        sems = (sem0, sem1)
        den_acc = jnp.zeros((16,), jnp.float32)
        h = pltpu.async_copy(L_hbm.at[r, pl.ds(0, _CH)], chunk0, sem0)
        for c in range(n_chunks):
            if c + 1 < n_chunks:
                h_next = pltpu.async_copy(
                    L_hbm.at[r, pl.ds((c + 1) * _CH, _CH)],
                    bufs[(c + 1) % 2], sems[(c + 1) % 2])
            h.wait()
            buf = bufs[c % 2]

            def vloop(v, dacc, buf=buf):
                idx_e = v * 32 + 2 * lane
                le = plsc.load_gather(buf, [idx_e])
                lo = plsc.load_gather(buf, [idx_e + 1])
                x = lo - le
                dacc = dacc + x * x
                binf = (x - _XMIN) * _INVW
                b = jnp.clip(binf.astype(jnp.int32), 0, _B - 1)
                addr = lane_base + b
                plsc.addupdate_scatter(s_cnt, [addr], ones)
                plsc.addupdate_scatter(s_sum, [addr], x)
                return dacc

            den_acc = lax.fori_loop(0, _CH // 32, vloop, den_acc, unroll=8)
            if c + 1 < n_chunks:
                h = h_next

        def red(j, _):
            off = j * 16
            ci = s_cnt[pl.ds(off, 16)]
            si = s_sum[pl.ds(off, 16)]
            for l in range(1, 16):
                ci = ci + s_cnt[pl.ds(l * _B + off, 16)]
                si = si + s_sum[pl.ds(l * _B + off, 16)]
            cnts[pl.ds(off, 16)] = ci
            sums[pl.ds(off, 16)] = si
            return 0
        lax.fori_loop(0, _B // 16, red, 0, unroll=2)

        def scan(j, carry):
            v = cnts[pl.ds(j * 16, 16)]
            inc = plsc.cumsum(v) + carry
            row_i = j // 8
            col = (j % 8) * 16
            cexc[row_i, pl.ds(col, 16)] = inc - v
            cinc[row_i, pl.ds(col, 16)] = inc
            return carry + jnp.sum(v)
        lax.fori_loop(0, _B // 16, scan, jnp.int32(0))

        handles = []
        for i in range(16):
            handles.append(pltpu.async_copy(Y_hbm.at[cexc.at[i]], yc.at[i], sem))
            handles.append(pltpu.async_copy(Y_hbm.at[cinc.at[i]], yt.at[i], sem))
        for h in handles:
            h.wait()

        def comb(j, acc):
            row_i = j // 8
            col = (j % 8) * 16
            t = cnts[pl.ds(j * 16, 16)]
            s = sums[pl.ds(j * 16, 16)]
            a = yc[row_i, pl.ds(col, 16)]
            bb = yt[row_i, pl.ds(col, 16)]
            tf = t.astype(jnp.float32)
            avg = s / jnp.maximum(tf, 1.0)
            return acc + jnp.where(t > 0, avg * (bb - a), 0.0)
        acc = lax.fori_loop(0, _B // 16, comb, jnp.zeros((16,), jnp.float32),
                            unroll=4)

        num_s = jnp.sum(acc)
        den_s = jnp.sum(den_acc)
        outv[...] = jnp.broadcast_to(num_s, (16,))
        pltpu.sync_copy(outv, num_hbm.at[r])
        outv[...] = jnp.broadcast_to(den_s, (16,))
        pltpu.sync_copy(outv, den_hbm.at[r])


def _sc_estimate(L, Y):
    mesh = plsc.VectorSubcoreMesh(core_axis_name="c", subcore_axis_name="s")
    f = pl.kernel(
        _sc_body,
        out_type=[
            jax.ShapeDtypeStruct((_ROWS, 16), jnp.float32),
            jax.ShapeDtypeStruct((_ROWS, 16), jnp.float32),
        ],
        mesh=mesh,
        compiler_params=pltpu.CompilerParams(needs_layout_passes=False),
        scratch_types=[
            pltpu.VMEM((_CH,), jnp.float32),
            pltpu.VMEM((_CH,), jnp.float32),
            pltpu.VMEM((_B * 16,), jnp.int32),
            pltpu.VMEM((_B * 16,), jnp.float32),
            pltpu.VMEM((_B,), jnp.int32),
            pltpu.VMEM((_B,), jnp.float32),
            pltpu.VMEM((16, _B // 16), jnp.int32),
            pltpu.VMEM((16, _B // 16), jnp.int32),
            pltpu.VMEM((16, _B // 16), jnp.float32),
            pltpu.VMEM((16, _B // 16), jnp.float32),
            pltpu.VMEM((16,), jnp.float32),
            pltpu.SemaphoreType.DMA,
            pltpu.SemaphoreType.DMA,
            pltpu.SemaphoreType.DMA,
        ],
    )
    return f(L, Y)


def kernel(distances):
    dflat = distances.reshape(_ROWS, 2 * _N)
    L = _log_all(dflat)
    Y = jnp.asarray(_Y_TABLE)
    num, den = _sc_estimate(L, Y)
    return num[:, 0] / den[:, 0]


# speedup vs baseline: 8.0307x; 1.2723x over previous
import functools

import numpy as np
import jax
import jax.numpy as jnp
from jax import lax
from jax.experimental import pallas as pl
from jax.experimental.pallas import tpu as pltpu
from jax.experimental.pallas import tpu_sc as plsc

_ROWS = 64
_N = 65536
_B = 2048
_XMIN = -6.9078
_XMAX = 6.9078
_INVW = _B / (_XMAX - _XMIN)
_NC = 2
_NS = 16
_CH = 8192

_yi = -np.log(1.0 - np.arange(_N, dtype=np.float64) / _N)
_Y_TABLE = np.concatenate([[0.0], np.cumsum(_yi)]).astype(np.float32)


def _log_body(d_ref, o_ref):
    o_ref[...] = jnp.log(d_ref[...])


def _log_all(dflat):
    return pl.pallas_call(
        _log_body,
        out_shape=jax.ShapeDtypeStruct(dflat.shape, jnp.float32),
        grid=(8, 4),
        in_specs=[pl.BlockSpec((8, 2 * _N // 4), lambda i, j: (i, j))],
        out_specs=pl.BlockSpec((8, 2 * _N // 4), lambda i, j: (i, j)),
    )(dflat)


def _sc_body(L_hbm, Y_hbm, num_hbm, den_hbm,
             chunk0, chunk1, s_cnt, s_sum, cnts, sums, cexc, cinc, yc, yt,
             outv, sem0, sem1, sem):
    wid = lax.axis_index("s") * _NC + lax.axis_index("c")
    lane = lax.iota(jnp.int32, 16)
    lane_base = lane * _B
    ones = jnp.ones((16,), jnp.int32)

    for rr in range(2):
        r = wid * 2 + rr

        @plsc.parallel_loop(0, _B, unroll=8)
        def _zero(i):
            s_cnt[pl.ds(i * 16, 16)] = jnp.zeros((16,), jnp.int32)
            s_sum[pl.ds(i * 16, 16)] = jnp.zeros((16,), jnp.float32)

        n_chunks = 2 * _N // _CH
        bufs = (chunk0, chunk1)
        sems = (sem0, sem1)
        den_acc = jnp.zeros((16,), jnp.float32)
        h = pltpu.async_copy(L_hbm.at[r, pl.ds(0, _CH)], chunk0, sem0)
        for c in range(n_chunks):
            if c + 1 < n_chunks:
                h_next = pltpu.async_copy(
                    L_hbm.at[r, pl.ds((c + 1) * _CH, _CH)],
                    bufs[(c + 1) % 2], sems[(c + 1) % 2])
            h.wait()
            buf = bufs[c % 2]

            @plsc.parallel_loop(0, _CH // 32, unroll=8, carry=den_acc)
            def vloop(v, dacc, buf=buf):
                idx_e = v * 32 + 2 * lane
                le = plsc.load_gather(buf, [idx_e])
                lo = plsc.load_gather(buf, [idx_e + 1])
                x = lo - le
                dacc = dacc + x * x
                binf = (x - _XMIN) * _INVW
                b = jnp.clip(binf.astype(jnp.int32), 0, _B - 1)
                addr = lane_base + b
                plsc.addupdate_scatter(s_cnt, [addr], ones)
                plsc.addupdate_scatter(s_sum, [addr], x)
                return dacc

            den_acc = vloop
            if c + 1 < n_chunks:
                h = h_next

        @plsc.parallel_loop(0, _B // 16, unroll=2)
        def _red(j):
            off = j * 16
            ci = s_cnt[pl.ds(off, 16)]
            si = s_sum[pl.ds(off, 16)]
            for l in range(1, 16):
                ci = ci + s_cnt[pl.ds(l * _B + off, 16)]
                si = si + s_sum[pl.ds(l * _B + off, 16)]
            cnts[pl.ds(off, 16)] = ci
            sums[pl.ds(off, 16)] = si

        @plsc.parallel_loop(0, _B // 16, unroll=4, carry=jnp.int32(0))
        def _scan(j, carry):
            v = cnts[pl.ds(j * 16, 16)]
            inc = plsc.cumsum(v) + carry
            row_i = j // 8
            col = (j % 8) * 16
            cexc[row_i, pl.ds(col, 16)] = inc - v
            cinc[row_i, pl.ds(col, 16)] = inc
            return carry + jnp.sum(v)

        handles = []
        for i in range(16):
            handles.append(pltpu.async_copy(Y_hbm.at[cexc.at[i]], yc.at[i], sem))
            handles.append(pltpu.async_copy(Y_hbm.at[cinc.at[i]], yt.at[i], sem))
        for h in handles:
            h.wait()

        @plsc.parallel_loop(0, _B // 16, unroll=4,
                            carry=jnp.zeros((16,), jnp.float32))
        def comb(j, acc):
            row_i = j // 8
            col = (j % 8) * 16
            t = cnts[pl.ds(j * 16, 16)]
            s = sums[pl.ds(j * 16, 16)]
            a = yc[row_i, pl.ds(col, 16)]
            bb = yt[row_i, pl.ds(col, 16)]
            tf = t.astype(jnp.float32)
            avg = s / jnp.maximum(tf, 1.0)
            return acc + jnp.where(t > 0, avg * (bb - a), 0.0)
        acc = comb

        num_s = jnp.sum(acc)
        den_s = jnp.sum(den_acc)
        outv[...] = jnp.broadcast_to(num_s, (16,))
        pltpu.sync_copy(outv, num_hbm.at[r])
        outv[...] = jnp.broadcast_to(den_s, (16,))
        pltpu.sync_copy(outv, den_hbm.at[r])


def _sc_estimate(L, Y):
    mesh = plsc.VectorSubcoreMesh(core_axis_name="c", subcore_axis_name="s")
    f = pl.kernel(
        _sc_body,
        out_type=[
            jax.ShapeDtypeStruct((_ROWS, 16), jnp.float32),
            jax.ShapeDtypeStruct((_ROWS, 16), jnp.float32),
        ],
        mesh=mesh,
        compiler_params=pltpu.CompilerParams(needs_layout_passes=False),
        scratch_types=[
            pltpu.VMEM((_CH,), jnp.float32),
            pltpu.VMEM((_CH,), jnp.float32),
            pltpu.VMEM((_B * 16,), jnp.int32),
            pltpu.VMEM((_B * 16,), jnp.float32),
            pltpu.VMEM((_B,), jnp.int32),
            pltpu.VMEM((_B,), jnp.float32),
            pltpu.VMEM((16, _B // 16), jnp.int32),
            pltpu.VMEM((16, _B // 16), jnp.int32),
            pltpu.VMEM((16, _B // 16), jnp.float32),
            pltpu.VMEM((16, _B // 16), jnp.float32),
            pltpu.VMEM((16,), jnp.float32),
            pltpu.SemaphoreType.DMA,
            pltpu.SemaphoreType.DMA,
            pltpu.SemaphoreType.DMA,
        ],
    )
    return f(L, Y)


def kernel(distances):
    dflat = distances.reshape(_ROWS, 2 * _N)
    L = _log_all(dflat)
    Y = jnp.asarray(_Y_TABLE)
    num, den = _sc_estimate(L, Y)
    return num[:, 0] / den[:, 0]
